# Initial kernel scaffold; baseline (speedup 1.0000x reference)
import functools

import jax
import jax.numpy as jnp
from jax import lax
from jax.experimental import pallas as pl
from jax.experimental.pallas import tpu as pltpu
from jax.experimental.pallas import tpu_sc as plsc

N_NODES = 10000
N_EDGES = 320000
D = 128

NC = 2
NS = 16
NW = NC * NS
E_PER_W = N_EDGES // NW
CHUNK = 100
NK = E_PER_W // CHUNK
ROWS_PER_TILE = N_NODES // NS


def _sc_segment_sum(x, src_w, dst_w, zeros):
    mesh = plsc.VectorSubcoreMesh(
        core_axis_name="c", subcore_axis_name="s", num_cores=NC, num_subcores=NS
    )

    @functools.partial(
        pl.kernel,
        out_type=jax.ShapeDtypeStruct((NC, N_NODES, D), jnp.float32),
        mesh=mesh,
        scratch_types=[
            pltpu.VMEM((NK, CHUNK), jnp.int32),
            pltpu.VMEM((NK, CHUNK), jnp.int32),
            pltpu.VMEM((CHUNK, D), jnp.float32),
            pltpu.VMEM_SHARED((N_NODES, D), jnp.float32),
            pltpu.SemaphoreType.DMA,
        ],
    )
    def agg_kernel(x_hbm, src_hbm, dst_hbm, zeros_hbm, out_hbm,
                   src_v, dst_v, rows_v, acc_sh, sem):
        c = lax.axis_index("c")
        s = lax.axis_index("s")
        wid = s * NC + c

        row0 = s * ROWS_PER_TILE
        pltpu.sync_copy(
            zeros_hbm.at[pl.ds(0, ROWS_PER_TILE)],
            acc_sh.at[pl.ds(row0, ROWS_PER_TILE)],
        )

        pltpu.sync_copy(src_hbm.at[wid], src_v)
        pltpu.sync_copy(dst_hbm.at[wid], dst_v)
        plsc.subcore_barrier()

        @pl.loop(0, NK)
        def body(j):
            pltpu.async_copy(x_hbm.at[src_v.at[j]], rows_v, sem).wait()
            pltpu.sync_copy(rows_v, acc_sh.at[dst_v.at[j]], add=True)

        plsc.subcore_barrier()
        pltpu.sync_copy(
            acc_sh.at[pl.ds(row0, ROWS_PER_TILE)],
            out_hbm.at[c].at[pl.ds(row0, ROWS_PER_TILE)],
        )

    return agg_kernel(x, src_w, dst_w, zeros)


def _dense_kernel(x_ref, a0_ref, a1_ref, ws_ref, wn_ref, b_ref, o_ref):
    agg = a0_ref[...] + a1_ref[...]
    h = jnp.dot(x_ref[...], ws_ref[...], preferred_element_type=jnp.float32)
    h = h + jnp.dot(agg, wn_ref[...], preferred_element_type=jnp.float32)
    o_ref[...] = jnp.maximum(h + b_ref[...], 0.0)


def _dense(x, a0, a1, W_self, W_nbr, b):
    blk = 2000
    grid = (N_NODES // blk,)
    return pl.pallas_call(
        _dense_kernel,
        out_shape=jax.ShapeDtypeStruct((N_NODES, D), jnp.float32),
        grid=grid,
        in_specs=[
            pl.BlockSpec((blk, D), lambda i: (i, 0)),
            pl.BlockSpec((blk, D), lambda i: (i, 0)),
            pl.BlockSpec((blk, D), lambda i: (i, 0)),
            pl.BlockSpec((D, D), lambda i: (0, 0)),
            pl.BlockSpec((D, D), lambda i: (0, 0)),
            pl.BlockSpec((1, D), lambda i: (0, 0)),
        ],
        out_specs=pl.BlockSpec((blk, D), lambda i: (i, 0)),
    )(x, a0, a1, W_self, W_nbr, b)


@jax.jit
def kernel(x, edge_index, W_self, W_nbr, b):
    ei = edge_index.astype(jnp.int32)
    src_w = ei[0].reshape(NW, NK, CHUNK)
    dst_w = ei[1].reshape(NW, NK, CHUNK)
    zeros = jnp.zeros((ROWS_PER_TILE, D), dtype=jnp.float32)
    agg = _sc_segment_sum(x, src_w, dst_w, zeros)
    return _dense(x, agg[0], agg[1], W_self, W_nbr, b.reshape(1, D))


# speedup vs baseline: 8.0604x; 8.0604x over previous
import functools

import jax
import jax.numpy as jnp
from jax import lax
from jax.experimental import pallas as pl
from jax.experimental.pallas import tpu as pltpu
from jax.experimental.pallas import tpu_sc as plsc

N_NODES = 10000
N_EDGES = 320000
D = 128

NC = 2
NS = 16
NW = NC * NS
E_PER_W = N_EDGES // NW
CHUNK = 100
NK = E_PER_W // CHUNK
N_PAD = 10240
ROWS_PER_TILE = N_PAD // NS


def _sc_segment_sum(x, src_w, dst_w, zeros):
    mesh = plsc.VectorSubcoreMesh(
        core_axis_name="c", subcore_axis_name="s", num_cores=NC, num_subcores=NS
    )

    @functools.partial(
        pl.kernel,
        out_type=jax.ShapeDtypeStruct((NC, N_PAD, D), jnp.float32),
        mesh=mesh,
        scratch_types=[
            pltpu.VMEM((NK, CHUNK), jnp.int32),
            pltpu.VMEM((NK, CHUNK), jnp.int32),
            pltpu.VMEM((CHUNK, D), jnp.float32),
            pltpu.VMEM_SHARED((N_PAD, D), jnp.float32),
            pltpu.SemaphoreType.DMA,
        ],
    )
    def agg_kernel(x_hbm, src_hbm, dst_hbm, zeros_hbm, out_hbm,
                   src_v, dst_v, rows_v, acc_sh, sem):
        c = lax.axis_index("c")
        s = lax.axis_index("s")
        wid = s * NC + c

        row0 = s * ROWS_PER_TILE
        pltpu.sync_copy(
            zeros_hbm.at[pl.ds(0, ROWS_PER_TILE)],
            acc_sh.at[pl.ds(row0, ROWS_PER_TILE)],
        )

        pltpu.sync_copy(src_hbm.at[wid], src_v)
        pltpu.sync_copy(dst_hbm.at[wid], dst_v)
        plsc.subcore_barrier()

        @pl.loop(0, NK)
        def body(j):
            pltpu.async_copy(x_hbm.at[src_v.at[j]], rows_v, sem).wait()
            pltpu.sync_copy(rows_v, acc_sh.at[dst_v.at[j]], add=True)

        plsc.subcore_barrier()
        pltpu.sync_copy(
            acc_sh.at[pl.ds(row0, ROWS_PER_TILE)],
            out_hbm.at[c].at[pl.ds(row0, ROWS_PER_TILE)],
        )

    return agg_kernel(x, src_w, dst_w, zeros)


def _dense_kernel(x_ref, a0_ref, a1_ref, ws_ref, wn_ref, b_ref, o_ref):
    agg = a0_ref[...] + a1_ref[...]
    h = jnp.dot(x_ref[...], ws_ref[...], preferred_element_type=jnp.float32)
    h = h + jnp.dot(agg, wn_ref[...], preferred_element_type=jnp.float32)
    o_ref[...] = jnp.maximum(h + b_ref[...], 0.0)


def _dense(x, a0, a1, W_self, W_nbr, b):
    blk = 2000
    grid = (N_NODES // blk,)
    return pl.pallas_call(
        _dense_kernel,
        out_shape=jax.ShapeDtypeStruct((N_NODES, D), jnp.float32),
        grid=grid,
        in_specs=[
            pl.BlockSpec((blk, D), lambda i: (i, 0)),
            pl.BlockSpec((blk, D), lambda i: (i, 0)),
            pl.BlockSpec((blk, D), lambda i: (i, 0)),
            pl.BlockSpec((D, D), lambda i: (0, 0)),
            pl.BlockSpec((D, D), lambda i: (0, 0)),
            pl.BlockSpec((1, D), lambda i: (0, 0)),
        ],
        out_specs=pl.BlockSpec((blk, D), lambda i: (i, 0)),
    )(x, a0, a1, W_self, W_nbr, b)


@jax.jit
def kernel(x, edge_index, W_self, W_nbr, b):
    ei = edge_index.astype(jnp.int32)
    src_w = ei[0].reshape(NW, NK, CHUNK)
    dst_w = ei[1].reshape(NW, NK, CHUNK)
    zeros = jnp.zeros((ROWS_PER_TILE, D), dtype=jnp.float32)
    agg = _sc_segment_sum(x, src_w, dst_w, zeros)
    return _dense(x, agg[0, :N_NODES], agg[1, :N_NODES], W_self, W_nbr,
                  b.reshape(1, D))
